# baseline (speedup 1.0000x reference)
import functools

import jax
import jax.numpy as jnp
import numpy as np
from jax.experimental import pallas as pl
from jax.experimental.pallas import tpu as pltpu

_K = 5
_IMG = 224
_BIG = 3.0e38


def _gauss_blur_matrix(size: int, sigma: float = 4.0) -> np.ndarray:
    radius = int(4.0 * sigma + 0.5)
    x = np.arange(-radius, radius + 1, dtype=np.float64)
    k = np.exp(-0.5 * (x / sigma) ** 2)
    k = k / k.sum()
    i = np.arange(size)
    d = i[None, :] - i[:, None] + radius
    valid = (d >= 0) & (d <= 2 * radius)
    return np.where(valid, k[np.clip(d, 0, 2 * radius)], 0.0)


def _bilinear_matrix(src: int, dst: int) -> np.ndarray:
    scale = dst / src
    sample = (np.arange(dst, dtype=np.float64) + 0.5) / scale - 0.5
    x = np.abs(sample[:, None] - np.arange(src, dtype=np.float64)[None, :])
    w = np.maximum(0.0, 1.0 - x)
    w = w / np.maximum(w.sum(axis=1, keepdims=True), 1e-12)
    return w


@functools.lru_cache(maxsize=None)
def _scale_matrices(src: int):
    m = _bilinear_matrix(src, _IMG)
    b = _gauss_blur_matrix(_IMG)
    left = (b @ m).astype(np.float32)
    right = (m.T @ b).astype(np.float32)
    return left, right


def _topk_body(zlib_ref, z_ref, idx_ref, score_ref):
    diff = zlib_ref[...] - z_ref[...]
    d2 = jnp.sum(diff * diff, axis=1, keepdims=True)
    ids = jax.lax.broadcasted_iota(jnp.int32, d2.shape, 0)
    acc = 0.0
    cur = d2
    for k in range(_K):
        m = jnp.min(cur)
        idx = jnp.min(jnp.where(cur == m, ids, 2**30))
        idx_ref[k] = idx
        acc = acc + jnp.sqrt(m)
        cur = jnp.where(ids == idx, _BIG, cur)
    score_ref[0] = acc / float(_K)


def _maps_body(idx_ref,
               l1_ref, f1_ref, l2_ref, f2_ref, l3_ref, f3_ref,
               a1_ref, b1_ref, a2_ref, b2_ref, a3_ref, b3_ref,
               out_ref, m1_ref, m2_ref, m3_ref):
    i = pl.program_id(0)

    def accum(l_ref, f_ref, m_ref):
        diff = l_ref[0] - f_ref[0]
        d2 = jnp.sum(diff * diff, axis=0)

        @pl.when(i == 0)
        def _():
            m_ref[...] = d2

        @pl.when(i > 0)
        def _():
            m_ref[...] = jnp.minimum(m_ref[...], d2)

    accum(l1_ref, f1_ref, m1_ref)
    accum(l2_ref, f2_ref, m2_ref)
    accum(l3_ref, f3_ref, m3_ref)

    @pl.when(i == _K - 1)
    def _():
        def up(a_ref, m_ref, b_ref):
            s = jnp.sqrt(m_ref[...])
            t = jnp.dot(a_ref[...], s, preferred_element_type=jnp.float32,
                        precision=jax.lax.Precision.HIGHEST)
            return jnp.dot(t, b_ref[...], preferred_element_type=jnp.float32,
                           precision=jax.lax.Precision.HIGHEST)

        out_ref[...] = (up(a1_ref, m1_ref, b1_ref)
                        + up(a2_ref, m2_ref, b2_ref)
                        + up(a3_ref, m3_ref, b3_ref))


def kernel(z_lib, z, fmap1, lib1, fmap2, lib2, fmap3, lib3):
    n = z_lib.shape[0]

    idx, score = pl.pallas_call(
        _topk_body,
        out_shape=(
            jax.ShapeDtypeStruct((_K,), jnp.int32),
            jax.ShapeDtypeStruct((1,), jnp.float32),
        ),
        out_specs=(
            pl.BlockSpec(memory_space=pltpu.SMEM),
            pl.BlockSpec(memory_space=pltpu.SMEM),
        ),
    )(z_lib, z)

    a1, b1 = _scale_matrices(fmap1.shape[-1])
    a2, b2 = _scale_matrices(fmap2.shape[-1])
    a3, b3 = _scale_matrices(fmap3.shape[-1])

    def lib_spec(c, h, w):
        return pl.BlockSpec((1, c, h, w), lambda i, idx: (idx[i], 0, 0, 0))

    def full_spec(c, h, w):
        return pl.BlockSpec((1, c, h, w), lambda i, idx: (0, 0, 0, 0))

    def mat_spec(r, c):
        return pl.BlockSpec((r, c), lambda i, idx: (0, 0))

    grid_spec = pltpu.PrefetchScalarGridSpec(
        num_scalar_prefetch=1,
        grid=(_K,),
        in_specs=[
            lib_spec(*lib1.shape[1:]), full_spec(*fmap1.shape[1:]),
            lib_spec(*lib2.shape[1:]), full_spec(*fmap2.shape[1:]),
            lib_spec(*lib3.shape[1:]), full_spec(*fmap3.shape[1:]),
            mat_spec(*a1.shape), mat_spec(*b1.shape),
            mat_spec(*a2.shape), mat_spec(*b2.shape),
            mat_spec(*a3.shape), mat_spec(*b3.shape),
        ],
        out_specs=pl.BlockSpec((_IMG, _IMG), lambda i, idx: (0, 0)),
        scratch_shapes=[
            pltpu.VMEM(fmap1.shape[2:], jnp.float32),
            pltpu.VMEM(fmap2.shape[2:], jnp.float32),
            pltpu.VMEM(fmap3.shape[2:], jnp.float32),
        ],
    )

    smap = pl.pallas_call(
        _maps_body,
        grid_spec=grid_spec,
        out_shape=jax.ShapeDtypeStruct((_IMG, _IMG), jnp.float32),
    )(idx, lib1, fmap1, lib2, fmap2, lib3, fmap3, a1, b1, a2, b2, a3, b3)

    return score[0], smap.reshape(1, 1, _IMG, _IMG)


# speedup vs baseline: 3.7652x; 3.7652x over previous
import functools

import jax
import jax.numpy as jnp
import numpy as np
from jax.experimental import pallas as pl
from jax.experimental.pallas import tpu as pltpu

_K = 5
_IMG = 224
_BIG = 3.0e38

_NP1, _CP1 = 28, 112
_NP2, _CP2 = 28, 28
_NP3, _CP3 = 14, 14


def _gauss_blur_matrix(size: int, sigma: float = 4.0) -> np.ndarray:
    radius = int(4.0 * sigma + 0.5)
    x = np.arange(-radius, radius + 1, dtype=np.float64)
    k = np.exp(-0.5 * (x / sigma) ** 2)
    k = k / k.sum()
    i = np.arange(size)
    d = i[None, :] - i[:, None] + radius
    valid = (d >= 0) & (d <= 2 * radius)
    return np.where(valid, k[np.clip(d, 0, 2 * radius)], 0.0)


def _bilinear_matrix(src: int, dst: int) -> np.ndarray:
    scale = dst / src
    sample = (np.arange(dst, dtype=np.float64) + 0.5) / scale - 0.5
    x = np.abs(sample[:, None] - np.arange(src, dtype=np.float64)[None, :])
    w = np.maximum(0.0, 1.0 - x)
    w = w / np.maximum(w.sum(axis=1, keepdims=True), 1e-12)
    return w


@functools.lru_cache(maxsize=None)
def _fold_matrices(src: int):
    m = _bilinear_matrix(src, _IMG)
    b = _gauss_blur_matrix(_IMG)
    left = b @ m
    right = m.T @ b
    pidx = np.arange(src * src)
    p_mat = left[:, pidx // src].astype(np.float32)
    q_mat = right[pidx % src, :].astype(np.float32)
    return p_mat, q_mat


def _ones_col(rows):
    i = jax.lax.broadcasted_iota(jnp.int32, (rows, 1), 0)
    return jnp.where(i >= 0, 1.0, 0.0).astype(jnp.float32)


def _body(zlib_ref, z_ref, l1_ref, f1_ref, l2_ref, f2_ref, l3_ref, f3_ref,
          p1_ref, q1_ref, p2_ref, q2_ref, p3_ref, q3_ref,
          score_ref, out_ref, s1_ref, s2_ref, s3_ref, mrow_ref, mcol_ref):
    i = pl.program_id(0)

    @pl.when(i == 0)
    def _():
        diff = zlib_ref[...] - z_ref[...]
        d2 = jnp.sum(diff * diff, axis=1, keepdims=True)
        vals = jnp.sqrt(d2)
        ids = jax.lax.broadcasted_iota(jnp.int32, vals.shape, 0)
        sel = []
        acc = 0.0
        cur = vals
        for _ in range(_K):
            m = jnp.min(cur)
            idx = jnp.min(jnp.where(cur == m, ids, 2**30))
            sel.append(idx)
            acc = acc + m
            cur = jnp.where(ids == idx, _BIG, cur)
        score_ref[0] = acc / float(_K)
        row_ids = jax.lax.broadcasted_iota(jnp.int32, (1, 200), 1)
        col_ids = jax.lax.broadcasted_iota(jnp.int32, (1, 200, 1), 1)
        mrow = jnp.full((1, 200), _BIG, jnp.float32)
        mcol = jnp.full((1, 200, 1), _BIG, jnp.float32)
        for idx in sel:
            mrow = jnp.where(row_ids == idx, 0.0, mrow)
            mcol = jnp.where(col_ids == idx, 0.0, mcol)
        mrow_ref[...] = mrow
        mcol_ref[...] = mcol

    @pl.when(i < _NP1)
    def _():
        c1 = l1_ref.shape[0]
        acc = None
        for c in range(c1):
            d = l1_ref[c] - f1_ref[:, c:c + 1]
            acc = d * d if acc is None else acc + d * d
        mn = jnp.min(acc + mrow_ref[...], axis=1, keepdims=True)
        off = pl.multiple_of(i * _CP1, _CP1)
        s1_ref[pl.ds(off, _CP1), :] = jnp.sqrt(mn)

    @pl.when(jnp.logical_and(i >= _NP1, i < _NP1 + _NP2))
    def _():
        d = l2_ref[...] - f2_ref[...]
        v = (d * d).reshape(_CP2 * 200, 128)
        dsq = jnp.dot(v, _ones_col(128),
                      preferred_element_type=jnp.float32,
                      precision=jax.lax.Precision.HIGHEST)
        d3 = dsq.reshape(_CP2, 200, 1) + mcol_ref[...]
        mn = jnp.min(d3, axis=1, keepdims=True)
        off = pl.multiple_of((i - _NP1) * _CP2, _CP2)
        s2_ref[pl.ds(off, _CP2), :] = jnp.sqrt(mn).reshape(_CP2, 1)

    @pl.when(jnp.logical_and(i >= _NP1 + _NP2, i < _NP1 + _NP2 + _NP3))
    def _():
        d = l3_ref[...] - f3_ref[...]
        v = (d * d).reshape(_CP3 * 200, 256)
        dsq = jnp.dot(v, _ones_col(256),
                      preferred_element_type=jnp.float32,
                      precision=jax.lax.Precision.HIGHEST)
        d3 = dsq.reshape(_CP3, 200, 1) + mcol_ref[...]
        mn = jnp.min(d3, axis=1, keepdims=True)
        off = pl.multiple_of((i - _NP1 - _NP2) * _CP3, _CP3)
        s3_ref[pl.ds(off, _CP3), :] = jnp.sqrt(mn).reshape(_CP3, 1)

    @pl.when(i == _NP1 + _NP2 + _NP3)
    def _():
        def up(p_ref, s_ref, q_ref):
            t = s_ref[...] * q_ref[...]
            return jnp.dot(p_ref[...], t, preferred_element_type=jnp.float32,
                           precision=jax.lax.Precision.HIGHEST)

        out_ref[...] = (up(p1_ref, s1_ref, q1_ref)
                        + up(p2_ref, s2_ref, q2_ref)
                        + up(p3_ref, s3_ref, q3_ref))


def kernel(z_lib, z, fmap1, lib1, fmap2, lib2, fmap3, lib3):
    n = z_lib.shape[0]
    c1, h1, w1 = lib1.shape[1:]
    c2, h2, w2 = lib2.shape[1:]
    c3, h3, w3 = lib3.shape[1:]
    hw1, hw2, hw3 = h1 * w1, h2 * w2, h3 * w3

    lib1_v = jnp.transpose(lib1, (1, 2, 3, 0)).reshape(c1, hw1, n)
    f1_v = jnp.transpose(fmap1, (0, 2, 3, 1)).reshape(hw1, c1)
    lib2_v = jnp.transpose(lib2, (2, 3, 0, 1)).reshape(hw2, n, c2)
    f2_v = jnp.transpose(fmap2, (2, 3, 0, 1)).reshape(hw2, 1, c2)
    lib3_v = jnp.transpose(lib3, (2, 3, 0, 1)).reshape(hw3, n, c3)
    f3_v = jnp.transpose(fmap3, (2, 3, 0, 1)).reshape(hw3, 1, c3)

    p1, q1 = _fold_matrices(w1)
    p2, q2 = _fold_matrices(w2)
    p3, q3 = _fold_matrices(w3)

    steps = _NP1 + _NP2 + _NP3 + 1

    def const2(r, c):
        return pl.BlockSpec((r, c), lambda i: (0, 0))

    score, smap = pl.pallas_call(
        _body,
        grid=(steps,),
        in_specs=[
            const2(n, z_lib.shape[1]),
            const2(1, z.shape[1]),
            pl.BlockSpec((c1, _CP1, n), lambda i: (0, jnp.clip(i, 0, _NP1 - 1), 0)),
            pl.BlockSpec((_CP1, c1), lambda i: (jnp.clip(i, 0, _NP1 - 1), 0)),
            pl.BlockSpec((_CP2, n, c2), lambda i: (jnp.clip(i - _NP1, 0, _NP2 - 1), 0, 0)),
            pl.BlockSpec((_CP2, 1, c2), lambda i: (jnp.clip(i - _NP1, 0, _NP2 - 1), 0, 0)),
            pl.BlockSpec((_CP3, n, c3), lambda i: (jnp.clip(i - _NP1 - _NP2, 0, _NP3 - 1), 0, 0)),
            pl.BlockSpec((_CP3, 1, c3), lambda i: (jnp.clip(i - _NP1 - _NP2, 0, _NP3 - 1), 0, 0)),
            const2(*p1.shape), const2(*q1.shape),
            const2(*p2.shape), const2(*q2.shape),
            const2(*p3.shape), const2(*q3.shape),
        ],
        out_specs=(
            pl.BlockSpec(memory_space=pltpu.SMEM),
            pl.BlockSpec((_IMG, _IMG), lambda i: (0, 0)),
        ),
        out_shape=(
            jax.ShapeDtypeStruct((1,), jnp.float32),
            jax.ShapeDtypeStruct((_IMG, _IMG), jnp.float32),
        ),
        scratch_shapes=[
            pltpu.VMEM((hw1, 1), jnp.float32),
            pltpu.VMEM((hw2, 1), jnp.float32),
            pltpu.VMEM((hw3, 1), jnp.float32),
            pltpu.VMEM((1, n), jnp.float32),
            pltpu.VMEM((1, n, 1), jnp.float32),
        ],
    )(z_lib, z, lib1_v, f1_v, lib2_v, f2_v, lib3_v, f3_v,
      p1, q1, p2, q2, p3, q3)

    return score[0], smap.reshape(1, 1, _IMG, _IMG)


# speedup vs baseline: 5.9734x; 1.5865x over previous
import functools

import jax
import jax.numpy as jnp
import numpy as np
from jax.experimental import pallas as pl
from jax.experimental.pallas import tpu as pltpu

_K = 5
_IMG = 224
_BIG = 3.0e38

_NP1, _CP1 = 28, 112
_NP2, _CP2 = 14, 56
_NP3, _CP3 = 7, 28


def _gauss_blur_matrix(size: int, sigma: float = 4.0) -> np.ndarray:
    radius = int(4.0 * sigma + 0.5)
    x = np.arange(-radius, radius + 1, dtype=np.float64)
    k = np.exp(-0.5 * (x / sigma) ** 2)
    k = k / k.sum()
    i = np.arange(size)
    d = i[None, :] - i[:, None] + radius
    valid = (d >= 0) & (d <= 2 * radius)
    return np.where(valid, k[np.clip(d, 0, 2 * radius)], 0.0)


def _bilinear_matrix(src: int, dst: int) -> np.ndarray:
    scale = dst / src
    sample = (np.arange(dst, dtype=np.float64) + 0.5) / scale - 0.5
    x = np.abs(sample[:, None] - np.arange(src, dtype=np.float64)[None, :])
    w = np.maximum(0.0, 1.0 - x)
    w = w / np.maximum(w.sum(axis=1, keepdims=True), 1e-12)
    return w


@functools.lru_cache(maxsize=None)
def _fold_matrices(src: int):
    m = _bilinear_matrix(src, _IMG)
    b = _gauss_blur_matrix(_IMG)
    left = b @ m
    right = m.T @ b
    pidx = np.arange(src * src)
    p_mat = left[:, pidx // src].astype(np.float32)
    q_mat = right[pidx % src, :].astype(np.float32)
    return p_mat, q_mat


def _ones_col(rows):
    i = jax.lax.broadcasted_iota(jnp.int32, (rows, 1), 0)
    return jnp.where(i >= 0, 1.0, 0.0).astype(jnp.float32)


def _body(zlib_ref, z_ref, l1_ref, f1_ref, l2_ref, f2_ref, l3_ref, f3_ref,
          p1_ref, q1_ref, p2_ref, q2_ref, p3_ref, q3_ref,
          score_ref, out_ref, s1_ref, s2_ref, s3_ref, mrow_ref, mcol_ref):
    i = pl.program_id(0)

    @pl.when(i == 0)
    def _():
        diff = zlib_ref[...] - z_ref[...]
        d2 = jnp.sum(diff * diff, axis=1, keepdims=True)
        vals = jnp.sqrt(d2)
        ids = jax.lax.broadcasted_iota(jnp.int32, vals.shape, 0)
        sel = []
        acc = 0.0
        cur = vals
        for _ in range(_K):
            m = jnp.min(cur)
            idx = jnp.min(jnp.where(cur == m, ids, 2**30))
            sel.append(idx)
            acc = acc + m
            cur = jnp.where(ids == idx, _BIG, cur)
        score_ref[0] = acc / float(_K)
        row_ids = jax.lax.broadcasted_iota(jnp.int32, (1, 200), 1)
        col_ids = jax.lax.broadcasted_iota(jnp.int32, (1, 200, 1), 1)
        mrow = jnp.full((1, 200), _BIG, jnp.float32)
        mcol = jnp.full((1, 200, 1), _BIG, jnp.float32)
        for idx in sel:
            mrow = jnp.where(row_ids == idx, 0.0, mrow)
            mcol = jnp.where(col_ids == idx, 0.0, mcol)
        mrow_ref[...] = mrow
        mcol_ref[...] = mcol

    @pl.when(i < _NP1)
    def _():
        c1 = l1_ref.shape[0]
        acc = None
        for c in range(c1):
            d = l1_ref[c] - f1_ref[:, c:c + 1]
            acc = d * d if acc is None else acc + d * d
        mn = jnp.min(acc + mrow_ref[...], axis=1, keepdims=True)
        off = pl.multiple_of(i * _CP1, _CP1)
        s1_ref[pl.ds(off, _CP1), :] = jnp.sqrt(mn)

    @pl.when(jnp.logical_and(i >= _NP1, i < _NP1 + _NP2))
    def _():
        d = l2_ref[...] - f2_ref[...]
        v = (d * d).reshape(_CP2 * 200, 128)
        dsq = jnp.dot(v, _ones_col(128),
                      preferred_element_type=jnp.float32)
        d3 = dsq.reshape(_CP2, 200, 1) + mcol_ref[...]
        mn = jnp.min(d3, axis=1, keepdims=True)
        off = pl.multiple_of((i - _NP1) * _CP2, _CP2)
        s2_ref[pl.ds(off, _CP2), :] = jnp.sqrt(mn).reshape(_CP2, 1)

    @pl.when(jnp.logical_and(i >= _NP1 + _NP2, i < _NP1 + _NP2 + _NP3))
    def _():
        d = l3_ref[...] - f3_ref[...]
        v = (d * d).reshape(_CP3 * 200, 256)
        dsq = jnp.dot(v, _ones_col(256),
                      preferred_element_type=jnp.float32)
        d3 = dsq.reshape(_CP3, 200, 1) + mcol_ref[...]
        mn = jnp.min(d3, axis=1, keepdims=True)
        off = pl.multiple_of((i - _NP1 - _NP2) * _CP3, _CP3)
        s3_ref[pl.ds(off, _CP3), :] = jnp.sqrt(mn).reshape(_CP3, 1)

    @pl.when(i == _NP1 + _NP2 + _NP3)
    def _():
        def up(p_ref, s_ref, q_ref):
            t = s_ref[...] * q_ref[...]
            return jnp.dot(p_ref[...], t, preferred_element_type=jnp.float32)

        out_ref[...] = (up(p1_ref, s1_ref, q1_ref)
                        + up(p2_ref, s2_ref, q2_ref)
                        + up(p3_ref, s3_ref, q3_ref))


def kernel(z_lib, z, fmap1, lib1, fmap2, lib2, fmap3, lib3):
    n = z_lib.shape[0]
    c1, h1, w1 = lib1.shape[1:]
    c2, h2, w2 = lib2.shape[1:]
    c3, h3, w3 = lib3.shape[1:]
    hw1, hw2, hw3 = h1 * w1, h2 * w2, h3 * w3

    lib1_v = jnp.transpose(lib1, (1, 2, 3, 0)).reshape(c1, hw1, n)
    f1_v = jnp.transpose(fmap1, (0, 2, 3, 1)).reshape(hw1, c1)
    lib2_v = jnp.transpose(lib2, (2, 3, 0, 1)).reshape(hw2, n, c2)
    f2_v = jnp.transpose(fmap2, (2, 3, 0, 1)).reshape(hw2, 1, c2)
    lib3_v = jnp.transpose(lib3, (2, 3, 0, 1)).reshape(hw3, n, c3)
    f3_v = jnp.transpose(fmap3, (2, 3, 0, 1)).reshape(hw3, 1, c3)

    p1, q1 = _fold_matrices(w1)
    p2, q2 = _fold_matrices(w2)
    p3, q3 = _fold_matrices(w3)

    steps = _NP1 + _NP2 + _NP3 + 1

    def const2(r, c):
        return pl.BlockSpec((r, c), lambda i: (0, 0))

    score, smap = pl.pallas_call(
        _body,
        grid=(steps,),
        in_specs=[
            const2(n, z_lib.shape[1]),
            const2(1, z.shape[1]),
            pl.BlockSpec((c1, _CP1, n), lambda i: (0, jnp.clip(i, 0, _NP1 - 1), 0)),
            pl.BlockSpec((_CP1, c1), lambda i: (jnp.clip(i, 0, _NP1 - 1), 0)),
            pl.BlockSpec((_CP2, n, c2), lambda i: (jnp.clip(i - _NP1, 0, _NP2 - 1), 0, 0)),
            pl.BlockSpec((_CP2, 1, c2), lambda i: (jnp.clip(i - _NP1, 0, _NP2 - 1), 0, 0)),
            pl.BlockSpec((_CP3, n, c3), lambda i: (jnp.clip(i - _NP1 - _NP2, 0, _NP3 - 1), 0, 0)),
            pl.BlockSpec((_CP3, 1, c3), lambda i: (jnp.clip(i - _NP1 - _NP2, 0, _NP3 - 1), 0, 0)),
            const2(*p1.shape), const2(*q1.shape),
            const2(*p2.shape), const2(*q2.shape),
            const2(*p3.shape), const2(*q3.shape),
        ],
        out_specs=(
            pl.BlockSpec(memory_space=pltpu.SMEM),
            pl.BlockSpec((_IMG, _IMG), lambda i: (0, 0)),
        ),
        out_shape=(
            jax.ShapeDtypeStruct((1,), jnp.float32),
            jax.ShapeDtypeStruct((_IMG, _IMG), jnp.float32),
        ),
        scratch_shapes=[
            pltpu.VMEM((hw1, 1), jnp.float32),
            pltpu.VMEM((hw2, 1), jnp.float32),
            pltpu.VMEM((hw3, 1), jnp.float32),
            pltpu.VMEM((1, n), jnp.float32),
            pltpu.VMEM((1, n, 1), jnp.float32),
        ],
    )(z_lib, z, lib1_v, f1_v, lib2_v, f2_v, lib3_v, f3_v,
      p1, q1, p2, q2, p3, q3)

    return score[0], smap.reshape(1, 1, _IMG, _IMG)


# speedup vs baseline: 9.4586x; 1.5834x over previous
import functools

import jax
import jax.numpy as jnp
import numpy as np
from jax.experimental import pallas as pl
from jax.experimental.pallas import tpu as pltpu

_K = 5
_IMG = 224
_BIG = 3.0e38

_NP1, _CP1 = 14, 224


def _gauss_blur_matrix(size: int, sigma: float = 4.0) -> np.ndarray:
    radius = int(4.0 * sigma + 0.5)
    x = np.arange(-radius, radius + 1, dtype=np.float64)
    k = np.exp(-0.5 * (x / sigma) ** 2)
    k = k / k.sum()
    i = np.arange(size)
    d = i[None, :] - i[:, None] + radius
    valid = (d >= 0) & (d <= 2 * radius)
    return np.where(valid, k[np.clip(d, 0, 2 * radius)], 0.0)


def _bilinear_matrix(src: int, dst: int) -> np.ndarray:
    scale = dst / src
    sample = (np.arange(dst, dtype=np.float64) + 0.5) / scale - 0.5
    x = np.abs(sample[:, None] - np.arange(src, dtype=np.float64)[None, :])
    w = np.maximum(0.0, 1.0 - x)
    w = w / np.maximum(w.sum(axis=1, keepdims=True), 1e-12)
    return w


@functools.lru_cache(maxsize=None)
def _fold_matrices(src: int):
    m = _bilinear_matrix(src, _IMG)
    b = _gauss_blur_matrix(_IMG)
    left = b @ m
    right = m.T @ b
    pidx = np.arange(src * src)
    p_mat = left[:, pidx // src].astype(np.float32)
    q_mat = right[pidx % src, :].astype(np.float32)
    return p_mat, q_mat


def _ones_col(rows):
    i = jax.lax.broadcasted_iota(jnp.int32, (rows, 1), 0)
    return jnp.where(i >= 0, 1.0, 0.0).astype(jnp.float32)


def _gather_copy(lib_hbm, buf_ref, sem, k, idx):
    return pltpu.make_async_copy(
        lib_hbm.at[:, pl.ds(idx, 1), :], buf_ref.at[k], sem.at[k])


def _body(zlib_ref, z_ref, l1_ref, f1_ref, l2_hbm, f2_ref, l3_hbm, f3_ref,
          p1_ref, q1_ref, p2_ref, q2_ref, p3_ref, q3_ref,
          score_ref, out_ref,
          s1_ref, mrow_ref, idx_ref, g2_ref, g3_ref, sem2, sem3):
    i = pl.program_id(0)

    @pl.when(i == 0)
    def _():
        diff = zlib_ref[...] - z_ref[...]
        d2 = jnp.sum(diff * diff, axis=1, keepdims=True)
        vals = jnp.sqrt(d2)
        ids = jax.lax.broadcasted_iota(jnp.int32, vals.shape, 0)
        sel = []
        acc = 0.0
        cur = vals
        for _ in range(_K):
            m = jnp.min(cur)
            idx = jnp.min(jnp.where(cur == m, ids, 2**30))
            sel.append(idx)
            acc = acc + m
            cur = jnp.where(ids == idx, _BIG, cur)
        score_ref[0] = acc / float(_K)
        row_ids = jax.lax.broadcasted_iota(jnp.int32, (1, 200), 1)
        mrow = jnp.full((1, 200), _BIG, jnp.float32)
        for k, idx in enumerate(sel):
            mrow = jnp.where(row_ids == idx, 0.0, mrow)
            idx_ref[k] = idx
            _gather_copy(l2_hbm, g2_ref, sem2, k, idx).start()
            _gather_copy(l3_hbm, g3_ref, sem3, k, idx).start()
        mrow_ref[...] = mrow

    @pl.when(i < _NP1)
    def _():
        c1 = l1_ref.shape[0]
        acc = None
        for c in range(c1):
            d = l1_ref[c] - f1_ref[:, c:c + 1]
            acc = d * d if acc is None else acc + d * d
        mn = jnp.min(acc + mrow_ref[...], axis=1, keepdims=True)
        off = pl.multiple_of(i * _CP1, _CP1)
        s1_ref[pl.ds(off, _CP1), :] = jnp.sqrt(mn)

    @pl.when(i == _NP1)
    def _():
        def min_maps(lib_hbm, g_ref, sem, f_ref, ones):
            mn = None
            for k in range(_K):
                _gather_copy(lib_hbm, g_ref, sem, k, idx_ref[k]).wait()
                d = g_ref[k, :, 0, :] - f_ref[...]
                dsq = jnp.dot(d * d, ones,
                              preferred_element_type=jnp.float32)
                mn = dsq if mn is None else jnp.minimum(mn, dsq)
            return jnp.sqrt(mn)

        s2 = min_maps(l2_hbm, g2_ref, sem2, f2_ref, _ones_col(128))
        s3 = min_maps(l3_hbm, g3_ref, sem3, f3_ref, _ones_col(256))

        def up(p_ref, s, q_ref):
            return jnp.dot(p_ref[...], s * q_ref[...],
                           preferred_element_type=jnp.float32)

        out_ref[...] = (up(p1_ref, s1_ref[...], q1_ref)
                        + up(p2_ref, s2, q2_ref)
                        + up(p3_ref, s3, q3_ref))


def kernel(z_lib, z, fmap1, lib1, fmap2, lib2, fmap3, lib3):
    n = z_lib.shape[0]
    c1, h1, w1 = lib1.shape[1:]
    c2, h2, w2 = lib2.shape[1:]
    c3, h3, w3 = lib3.shape[1:]
    hw1, hw2, hw3 = h1 * w1, h2 * w2, h3 * w3

    lib1_v = jnp.transpose(lib1, (1, 2, 3, 0)).reshape(c1, hw1, n)
    f1_v = jnp.transpose(fmap1, (0, 2, 3, 1)).reshape(hw1, c1)
    lib2_v = jnp.transpose(lib2, (2, 3, 0, 1)).reshape(hw2, n, c2)
    f2_v = jnp.transpose(fmap2, (2, 3, 0, 1)).reshape(hw2, c2)
    lib3_v = jnp.transpose(lib3, (2, 3, 0, 1)).reshape(hw3, n, c3)
    f3_v = jnp.transpose(fmap3, (2, 3, 0, 1)).reshape(hw3, c3)

    p1, q1 = _fold_matrices(w1)
    p2, q2 = _fold_matrices(w2)
    p3, q3 = _fold_matrices(w3)

    steps = _NP1 + 1

    def const2(r, c):
        return pl.BlockSpec((r, c), lambda i: (0, 0))

    score, smap = pl.pallas_call(
        _body,
        grid=(steps,),
        in_specs=[
            const2(n, z_lib.shape[1]),
            const2(1, z.shape[1]),
            pl.BlockSpec((c1, _CP1, n), lambda i: (0, jnp.clip(i, 0, _NP1 - 1), 0)),
            pl.BlockSpec((_CP1, c1), lambda i: (jnp.clip(i, 0, _NP1 - 1), 0)),
            pl.BlockSpec(memory_space=pl.ANY),
            const2(hw2, c2),
            pl.BlockSpec(memory_space=pl.ANY),
            const2(hw3, c3),
            const2(*p1.shape), const2(*q1.shape),
            const2(*p2.shape), const2(*q2.shape),
            const2(*p3.shape), const2(*q3.shape),
        ],
        out_specs=(
            pl.BlockSpec(memory_space=pltpu.SMEM),
            pl.BlockSpec((_IMG, _IMG), lambda i: (0, 0)),
        ),
        out_shape=(
            jax.ShapeDtypeStruct((1,), jnp.float32),
            jax.ShapeDtypeStruct((_IMG, _IMG), jnp.float32),
        ),
        scratch_shapes=[
            pltpu.VMEM((hw1, 1), jnp.float32),
            pltpu.VMEM((1, n), jnp.float32),
            pltpu.SMEM((_K,), jnp.int32),
            pltpu.VMEM((_K, hw2, 1, c2), jnp.float32),
            pltpu.VMEM((_K, hw3, 1, c3), jnp.float32),
            pltpu.SemaphoreType.DMA((_K,)),
            pltpu.SemaphoreType.DMA((_K,)),
        ],
    )(z_lib, z, lib1_v, f1_v, lib2_v, f2_v, lib3_v, f3_v,
      p1, q1, p2, q2, p3, q3)

    return score[0], smap.reshape(1, 1, _IMG, _IMG)


# speedup vs baseline: 9.5964x; 1.0146x over previous
import functools

import jax
import jax.numpy as jnp
import numpy as np
from jax.experimental import pallas as pl
from jax.experimental.pallas import tpu as pltpu

_K = 5
_IMG = 224
_BIG = 3.0e38

_NP1, _CP1 = 14, 224


def _gauss_blur_matrix(size: int, sigma: float = 4.0) -> np.ndarray:
    radius = int(4.0 * sigma + 0.5)
    x = np.arange(-radius, radius + 1, dtype=np.float64)
    k = np.exp(-0.5 * (x / sigma) ** 2)
    k = k / k.sum()
    i = np.arange(size)
    d = i[None, :] - i[:, None] + radius
    valid = (d >= 0) & (d <= 2 * radius)
    return np.where(valid, k[np.clip(d, 0, 2 * radius)], 0.0)


def _bilinear_matrix(src: int, dst: int) -> np.ndarray:
    scale = dst / src
    sample = (np.arange(dst, dtype=np.float64) + 0.5) / scale - 0.5
    x = np.abs(sample[:, None] - np.arange(src, dtype=np.float64)[None, :])
    w = np.maximum(0.0, 1.0 - x)
    w = w / np.maximum(w.sum(axis=1, keepdims=True), 1e-12)
    return w


@functools.lru_cache(maxsize=None)
def _fold_matrices(src: int):
    m = _bilinear_matrix(src, _IMG)
    b = _gauss_blur_matrix(_IMG)
    left = b @ m
    right = m.T @ b
    pidx = np.arange(src * src)
    p_mat = left[:, pidx // src].astype(np.float32)
    q_mat = right[pidx % src, :].astype(np.float32)
    return p_mat, q_mat


def _ones_col(rows):
    i = jax.lax.broadcasted_iota(jnp.int32, (rows, 1), 0)
    return jnp.where(i >= 0, 1.0, 0.0).astype(jnp.float32)


def _gather_copy(lib_hbm, buf_ref, sem, k, idx):
    return pltpu.make_async_copy(
        lib_hbm.at[:, idx, :], buf_ref.at[k], sem.at[k])


def _body(zlib_ref, z_ref, l1_ref, f1_ref, l2_hbm, f2_ref, l3_hbm, f3_ref,
          p1_ref, q1_ref, p2_ref, q2_ref, p3_ref, q3_ref,
          score_ref, out_ref,
          s1_ref, mrow_ref, idx_ref, g2_ref, g3_ref, sem2, sem3):
    i = pl.program_id(0)

    @pl.when(i == 0)
    def _():
        diff = zlib_ref[...] - z_ref[...]
        d2 = jnp.sum(diff * diff, axis=1, keepdims=True)
        vals = jnp.sqrt(d2)
        ids = jax.lax.broadcasted_iota(jnp.int32, vals.shape, 0)
        sel = []
        acc = 0.0
        cur = vals
        for _ in range(_K):
            m = jnp.min(cur)
            idx = jnp.min(jnp.where(cur == m, ids, 2**30))
            sel.append(idx)
            acc = acc + m
            cur = jnp.where(ids == idx, _BIG, cur)
        score_ref[0] = acc / float(_K)
        row_ids = jax.lax.broadcasted_iota(jnp.int32, (1, 200), 1)
        mrow = jnp.full((1, 200), _BIG, jnp.float32)
        for k, idx in enumerate(sel):
            mrow = jnp.where(row_ids == idx, 0.0, mrow)
            idx_ref[k] = idx
            _gather_copy(l2_hbm, g2_ref, sem2, k, idx).start()
            _gather_copy(l3_hbm, g3_ref, sem3, k, idx).start()
        mrow_ref[...] = mrow

    @pl.when(i < _NP1)
    def _():
        c1 = l1_ref.shape[0]
        acc = None
        for c in range(c1):
            d = l1_ref[c] - f1_ref[:, c:c + 1]
            acc = d * d if acc is None else acc + d * d
        mn = jnp.min(acc + mrow_ref[...], axis=1, keepdims=True)
        off = pl.multiple_of(i * _CP1, _CP1)
        s1_ref[pl.ds(off, _CP1), :] = jnp.sqrt(mn)

    @pl.when(i == _NP1)
    def _():
        def min_maps(lib_hbm, g_ref, sem, f_ref, ones):
            for k in range(_K):
                _gather_copy(lib_hbm, g_ref, sem, k, idx_ref[k]).wait()
            hw, c = f_ref.shape
            d = g_ref[...] - f_ref[...][None]
            dsq = jnp.dot((d * d).reshape(_K * hw, c), ones,
                          preferred_element_type=jnp.float32)
            mn = jnp.min(dsq.reshape(_K, hw, 1), axis=0)
            return jnp.sqrt(mn)

        s2 = min_maps(l2_hbm, g2_ref, sem2, f2_ref, _ones_col(128))
        s3 = min_maps(l3_hbm, g3_ref, sem3, f3_ref, _ones_col(256))

        def up(p_ref, s, q_ref):
            return jnp.dot(p_ref[...], s * q_ref[...],
                           preferred_element_type=jnp.float32)

        out_ref[...] = (up(p1_ref, s1_ref[...], q1_ref)
                        + up(p2_ref, s2, q2_ref)
                        + up(p3_ref, s3, q3_ref))


def kernel(z_lib, z, fmap1, lib1, fmap2, lib2, fmap3, lib3):
    n = z_lib.shape[0]
    c1, h1, w1 = lib1.shape[1:]
    c2, h2, w2 = lib2.shape[1:]
    c3, h3, w3 = lib3.shape[1:]
    hw1, hw2, hw3 = h1 * w1, h2 * w2, h3 * w3

    lib1_v = jnp.transpose(lib1, (1, 2, 3, 0)).reshape(c1, hw1, n)
    f1_v = jnp.transpose(fmap1, (0, 2, 3, 1)).reshape(hw1, c1)
    lib2_v = jnp.transpose(lib2, (2, 3, 0, 1)).reshape(hw2, n, c2)
    f2_v = jnp.transpose(fmap2, (2, 3, 0, 1)).reshape(hw2, c2)
    lib3_v = jnp.transpose(lib3, (2, 3, 0, 1)).reshape(hw3, n, c3)
    f3_v = jnp.transpose(fmap3, (2, 3, 0, 1)).reshape(hw3, c3)

    p1, q1 = _fold_matrices(w1)
    p2, q2 = _fold_matrices(w2)
    p3, q3 = _fold_matrices(w3)

    steps = _NP1 + 1

    def const2(r, c):
        return pl.BlockSpec((r, c), lambda i: (0, 0))

    score, smap = pl.pallas_call(
        _body,
        grid=(steps,),
        in_specs=[
            const2(n, z_lib.shape[1]),
            const2(1, z.shape[1]),
            pl.BlockSpec((c1, _CP1, n), lambda i: (0, jnp.clip(i, 0, _NP1 - 1), 0)),
            pl.BlockSpec((_CP1, c1), lambda i: (jnp.clip(i, 0, _NP1 - 1), 0)),
            pl.BlockSpec(memory_space=pl.ANY),
            const2(hw2, c2),
            pl.BlockSpec(memory_space=pl.ANY),
            const2(hw3, c3),
            const2(*p1.shape), const2(*q1.shape),
            const2(*p2.shape), const2(*q2.shape),
            const2(*p3.shape), const2(*q3.shape),
        ],
        out_specs=(
            pl.BlockSpec(memory_space=pltpu.SMEM),
            pl.BlockSpec((_IMG, _IMG), lambda i: (0, 0)),
        ),
        out_shape=(
            jax.ShapeDtypeStruct((1,), jnp.float32),
            jax.ShapeDtypeStruct((_IMG, _IMG), jnp.float32),
        ),
        scratch_shapes=[
            pltpu.VMEM((hw1, 1), jnp.float32),
            pltpu.VMEM((1, n), jnp.float32),
            pltpu.SMEM((_K,), jnp.int32),
            pltpu.VMEM((_K, hw2, c2), jnp.float32),
            pltpu.VMEM((_K, hw3, c3), jnp.float32),
            pltpu.SemaphoreType.DMA((_K,)),
            pltpu.SemaphoreType.DMA((_K,)),
        ],
    )(z_lib, z, lib1_v, f1_v, lib2_v, f2_v, lib3_v, f3_v,
      p1, q1, p2, q2, p3, q3)

    return score[0], smap.reshape(1, 1, _IMG, _IMG)
